# baseline (speedup 1.0000x reference)
import functools
import jax
import jax.numpy as jnp
from jax.experimental import pallas as pl
from jax.experimental.pallas import tpu as pltpu

H = 8
C = 64
D = H * C
NS = 2048
NT = 4096


def _proj_kernel(x_ref, w_ref, o_ref):
    o_ref[...] = jax.lax.dot(x_ref[...], w_ref[...],
                             preferred_element_type=jnp.float32)


def _proj(x, wcat):
    return pl.pallas_call(
        _proj_kernel,
        grid=(4,),
        in_specs=[
            pl.BlockSpec((NS, NS), lambda j: (0, 0)),
            pl.BlockSpec((NS, D), lambda j: (0, j)),
        ],
        out_specs=pl.BlockSpec((NS, D), lambda j: (0, j)),
        out_shape=jax.ShapeDtypeStruct((NS, 4 * D), jnp.float32),
    )(x, wcat)


def _norm_kernel(out1_ref, r_ref, gw_ref, gb_ref, gms_ref, h_ref):
    out = out1_ref[...] + r_ref[...]
    mean = jnp.mean(out, axis=0, keepdims=True)
    o = out - mean * gms_ref[...]
    var = jnp.mean(o * o, axis=0, keepdims=True)
    o = gw_ref[...] * o * jax.lax.rsqrt(var + 1e-5) + gb_ref[...]
    h_ref[...] = jnp.maximum(o, 0.0)


def _norm(out1, r, gn_w, gn_b, gn_ms):
    return pl.pallas_call(
        _norm_kernel,
        in_specs=[
            pl.BlockSpec((NS, D), lambda: (0, 0)),
            pl.BlockSpec((NS, D), lambda: (0, 0)),
            pl.BlockSpec((1, D), lambda: (0, 0)),
            pl.BlockSpec((1, D), lambda: (0, 0)),
            pl.BlockSpec((1, D), lambda: (0, 0)),
        ],
        out_specs=pl.BlockSpec((NS, D), lambda: (0, 0)),
        out_shape=jax.ShapeDtypeStruct((NS, D), jnp.float32),
    )(out1, r, gn_w.reshape(1, D), gn_b.reshape(1, D), gn_ms.reshape(1, D))


def _bip_kernel(w_ref, h_ref, b_ref, o_ref):
    o_ref[...] = jax.lax.dot(w_ref[...], h_ref[...],
                             preferred_element_type=jnp.float32) + b_ref[...]


def _bip(Wbip, h, bbip):
    return pl.pallas_call(
        _bip_kernel,
        grid=(8,),
        in_specs=[
            pl.BlockSpec((512, NS), lambda i: (i, 0)),
            pl.BlockSpec((NS, D), lambda i: (0, 0)),
            pl.BlockSpec((512, 1), lambda i: (i, 0)),
        ],
        out_specs=pl.BlockSpec((512, D), lambda i: (i, 0)),
        out_shape=jax.ShapeDtypeStruct((NT, D), jnp.float32),
    )(Wbip, h, bbip.reshape(NT, 1))


_TB = 512
_NTB = NT // _TB


def _xtm_kernel(a_ref, b_ref, o_ref, mn_ref, mx_ref):
    t = jax.lax.dot_general(a_ref[...], b_ref[...],
                            (((1,), (1,)), ((), ())),
                            preferred_element_type=jnp.float32)
    o_ref[...] = t
    mn_ref[...] = jnp.broadcast_to(jnp.min(t), (1, 1, 128))
    mx_ref[...] = jnp.broadcast_to(jnp.max(t), (1, 1, 128))


def _xtm(xt):
    return pl.pallas_call(
        _xtm_kernel,
        grid=(_NTB, _NTB),
        in_specs=[
            pl.BlockSpec((_TB, D), lambda i, j: (i, 0)),
            pl.BlockSpec((_TB, D), lambda i, j: (j, 0)),
        ],
        out_specs=[
            pl.BlockSpec((_TB, _TB), lambda i, j: (i, j)),
            pl.BlockSpec((1, 1, 128), lambda i, j: (i * _NTB + j, 0, 0)),
            pl.BlockSpec((1, 1, 128), lambda i, j: (i * _NTB + j, 0, 0)),
        ],
        out_shape=[
            jax.ShapeDtypeStruct((NT, NT), jnp.float32),
            jax.ShapeDtypeStruct((_NTB * _NTB, 1, 128), jnp.float32),
            jax.ShapeDtypeStruct((_NTB * _NTB, 1, 128), jnp.float32),
        ],
    )(xt, xt)


def _final_kernel(x_ref, mn_ref, mx_ref, o_ref):
    mn = jnp.min(mn_ref[...])
    mx = jnp.max(mx_ref[...])
    o_ref[...] = (x_ref[...] - mn) / (mx - mn + 1e-8)


def _final(xtm, mns, mxs):
    return pl.pallas_call(
        _final_kernel,
        grid=(_NTB,),
        in_specs=[
            pl.BlockSpec((_TB, NT), lambda i: (i, 0)),
            pl.BlockSpec((_NTB * _NTB, 1, 128), lambda i: (0, 0, 0)),
            pl.BlockSpec((_NTB * _NTB, 1, 128), lambda i: (0, 0, 0)),
        ],
        out_specs=pl.BlockSpec((_TB, NT), lambda i: (i, 0)),
        out_shape=jax.ShapeDtypeStruct((NT, NT), jnp.float32),
    )(xtm, mns, mxs)


def _edge_phase(q, k, v, edge_attr, We, be, src, dst):
    e = (edge_attr @ We.T + be).reshape(-1, H, C)
    kj = k[src] + e
    vj = v[src] + e
    qi = q[dst]
    alpha = (qi * kj).sum(-1) / jnp.sqrt(jnp.float32(C))
    amax = jax.ops.segment_max(alpha, dst, num_segments=NS)
    amax = jnp.where(jnp.isfinite(amax), amax, 0.0)
    ex = jnp.exp(alpha - amax[dst])
    den = jax.ops.segment_sum(ex, dst, num_segments=NS)
    a = ex / (den[dst] + 1e-16)
    out = jax.ops.segment_sum(vj * a[:, :, None], dst,
                              num_segments=NS).reshape(NS, D)
    return out


def kernel(x, pos_edge_index, edge_attr, Wq, bq, Wk, bk, Wv, bv, We, be,
           Wr, br, gn_w, gn_b, gn_ms, Wbip, bbip):
    src = pos_edge_index[0]
    dst = pos_edge_index[1]
    wcat = jnp.concatenate([Wq.T, Wk.T, Wv.T, Wr.T], axis=1)
    qkvr = _proj(x, wcat)
    q = qkvr[:, :D].reshape(NS, H, C)
    k = qkvr[:, D:2 * D].reshape(NS, H, C)
    v = qkvr[:, 2 * D:3 * D].reshape(NS, H, C)
    r = qkvr[:, 3 * D:]

    out1 = _edge_phase(q, k, v, edge_attr, We, be, src, dst)

    h = _norm(out1, r, gn_w, gn_b, gn_ms)
    xt = _bip(Wbip, h, bbip)
    xtm, mns, mxs = _xtm(xt)
    return _final(xtm, mns, mxs)
